# baseline (speedup 1.0000x reference)
import functools

import jax
import jax.numpy as jnp
from jax import lax
from jax.experimental import pallas as pl
from jax.experimental.pallas import tpu as pltpu
from jax.experimental.pallas import tpu_sc as plsc

TERMS = 1000000
D = 16
NC = 2
NS = 16
NW = NC * NS
CHUNK = 1024
NBUF = 3
BLK = 8192


def _tc_clamp_t(Wt):
    grid = (TERMS + BLK - 1) // BLK

    def body(w_ref, o_ref):
        t = jnp.clip(w_ref[...].T, 0.0, 1.0).reshape(BLK // 2, 2, D)
        for e in range(2):
            o_ref[:, e * D:(e + 1) * D] = t[:, e, :]

    return pl.pallas_call(
        body,
        grid=(grid,),
        in_specs=[pl.BlockSpec((D, BLK), lambda i: (0, i))],
        out_specs=pl.BlockSpec((BLK // 2, 128), lambda i: (i, 0)),
        out_shape=jax.ShapeDtypeStruct((TERMS // 2, 128), jnp.float32),
    )(Wt)


def _sc_gather_t(xf, Wc, n):
    rows_per_w = n // NW
    n_chunks = rows_per_w // CHUNK
    nh = n // 16384
    gpc = CHUNK // 128

    mesh = plsc.VectorSubcoreMesh(core_axis_name="c", subcore_axis_name="s")

    @functools.partial(
        pl.kernel,
        mesh=mesh,
        compiler_params=pltpu.CompilerParams(
            use_tc_tiling_on_sc=False, needs_layout_passes=False),
        out_type=jax.ShapeDtypeStruct((nh, 2, 128, 8, 128), jnp.float32),
        scratch_types=[
            pltpu.VMEM((NBUF, CHUNK), jnp.int32),
            pltpu.VMEM((NBUF, CHUNK, D), jnp.float32),
            pltpu.VMEM((NBUF, gpc, 2, 8, 128), jnp.float32),
        ]
        + [pltpu.SemaphoreType.DMA] * (3 * NBUF),
    )
    def k(x_hbm, w_hbm, out_hbm, idx_v, rows_v, t_v, *sems):
        sem_i = sems[0:NBUF]
        sem_g = sems[NBUF:2 * NBUF]
        sem_o = sems[2 * NBUF:3 * NBUF]
        wid = lax.axis_index("s") * NC + lax.axis_index("c")
        base = wid * rows_per_w
        lanes = lax.iota(jnp.int32, 16)

        def fire_idx(ci, b):
            pltpu.async_copy(
                x_hbm.at[pl.ds(base + ci * CHUNK, CHUNK)],
                idx_v.at[b], sem_i[b])

        def drain_idx(ci, b):
            pltpu.make_async_copy(
                x_hbm.at[pl.ds(base + ci * CHUNK, CHUNK)],
                idx_v.at[b], sem_i[b]).wait()

        def scale_idx(b):
            @plsc.parallel_loop(0, CHUNK // 16, unroll=4)
            def _(j):
                off = j << 4
                idx_v[b, pl.ds(off, 16)] = idx_v[b, pl.ds(off, 16)] << 2

        def fire_gather(ci, b):
            pltpu.async_copy(w_hbm.at[idx_v.at[b]], rows_v.at[b], sem_g[b])

        def drain_gather(ci, b):
            pltpu.make_async_copy(
                w_hbm.at[idx_v.at[b]], rows_v.at[b], sem_g[b]).wait()

        def _out_slices(ci, b):
            tau = (base >> 10) + ci
            ht = tau >> 7
            bt = tau & 127
            pairs = []
            for hi in range(gpc):
                for dt in range(2):
                    pairs.append((t_v.at[b, hi, dt],
                                  out_hbm.at[ht * 8 + hi, dt, bt]))
            return pairs

        def fire_out(ci, b):
            for src, dst in _out_slices(ci, b):
                pltpu.async_copy(src, dst, sem_o[b])

        def drain_out(ci, b):
            for src, dst in _out_slices(ci, b):
                pltpu.make_async_copy(src, dst, sem_o[b]).wait()

        col_vecs = [jnp.full((16,), d, jnp.int32) for d in range(D)]

        def transpose_chunk(b):
            g_ref = rows_v.at[b]

            @plsc.parallel_loop(0, CHUNK // 16, unroll=2)
            def _(j):
                rows = (j << 4) + lanes
                hi = j >> 3
                bi0 = (j & 7) << 4
                for d in range(D):
                    v = plsc.load_gather(g_ref, [rows, col_vecs[d]])
                    t_v[b, hi, d >> 3, d & 7, pl.ds(bi0, 16)] = v

        fire_idx(0, 0)
        fire_idx(1, 1)
        drain_idx(0, 0)
        scale_idx(0)
        fire_gather(0, 0)

        n_iters = n_chunks + 2
        assert n_iters % NBUF == 0

        def ring_body(c0):
            for u in range(NBUF):
                ci = c0 + u
                b0 = u
                b1 = (u + 1) % NBUF
                b2 = (u + 2) % NBUF

                @pl.when(ci + 2 < n_chunks)
                def _():
                    fire_idx(ci + 2, b2)

                @pl.when(ci + 1 < n_chunks)
                def _():
                    drain_idx(ci + 1, b1)
                    scale_idx(b1)
                    fire_gather(ci + 1, b1)

                @pl.when(ci >= 2)
                def _():
                    drain_out(ci - 2, b1)

                @pl.when(ci < n_chunks)
                def _():
                    drain_gather(ci, b0)
                    transpose_chunk(b0)
                    fire_out(ci, b0)

        pl.loop(0, n_iters, step=NBUF, unroll=False)(ring_body)

    return k(xf, Wc)


def kernel(x, W):
    b, h = x.shape
    n = b * h
    x4 = x.reshape(b // 128, 128, h // 8, 8)
    xf = jnp.transpose(x4, (2, 0, 3, 1)).reshape(n).astype(jnp.int32)
    wc = _tc_clamp_t(jnp.transpose(W)).reshape(TERMS * 4, D)
    p5 = _sc_gather_t(xf, wc, n)
    return jnp.transpose(p5, (2, 4, 0, 1, 3)).reshape(b, h, D)


# speedup vs baseline: 1.3669x; 1.3669x over previous
import functools

import jax
import jax.numpy as jnp
from jax import lax
from jax.experimental import pallas as pl
from jax.experimental.pallas import tpu as pltpu
from jax.experimental.pallas import tpu_sc as plsc

TERMS = 1000000
D = 16
NC = 2
NS = 16
NW = NC * NS
CHUNK = 1024
NBUF = 3
BLK = 8192


def _tc_clamp_t(Wt):
    grid = (TERMS + BLK - 1) // BLK

    def body(w_ref, o_ref):
        o_ref[:, :D] = jnp.clip(w_ref[...].T, 0.0, 1.0)

    return pl.pallas_call(
        body,
        grid=(grid,),
        in_specs=[pl.BlockSpec((D, BLK), lambda i: (0, i))],
        out_specs=pl.BlockSpec((BLK, 128), lambda i: (i, 0)),
        out_shape=jax.ShapeDtypeStruct((TERMS, 128), jnp.float32),
    )(Wt)


def _sc_gather_t(xf, Wc, n):
    rows_per_w = n // NW
    n_chunks = rows_per_w // CHUNK
    nh = n // 16384
    gpc = CHUNK // 128

    mesh = plsc.VectorSubcoreMesh(core_axis_name="c", subcore_axis_name="s")

    @functools.partial(
        pl.kernel,
        mesh=mesh,
        compiler_params=pltpu.CompilerParams(
            use_tc_tiling_on_sc=False, needs_layout_passes=False),
        out_type=jax.ShapeDtypeStruct((nh, 2, 128, 8, 128), jnp.float32),
        scratch_types=[
            pltpu.VMEM((NBUF, CHUNK), jnp.int32),
            pltpu.VMEM((NBUF, CHUNK, D), jnp.float32),
            pltpu.VMEM((NBUF, gpc, 2, 8, 128), jnp.float32),
        ]
        + [pltpu.SemaphoreType.DMA] * (3 * NBUF),
    )
    def k(x_hbm, w_hbm, out_hbm, idx_v, rows_v, t_v, *sems):
        sem_i = sems[0:NBUF]
        sem_g = sems[NBUF:2 * NBUF]
        sem_o = sems[2 * NBUF:3 * NBUF]
        wid = lax.axis_index("s") * NC + lax.axis_index("c")
        base = wid * rows_per_w
        lanes = lax.iota(jnp.int32, 16)

        def fire_idx(ci, b):
            pltpu.async_copy(
                x_hbm.at[pl.ds(base + ci * CHUNK, CHUNK)],
                idx_v.at[b], sem_i[b])

        def drain_idx(ci, b):
            pltpu.make_async_copy(
                x_hbm.at[pl.ds(base + ci * CHUNK, CHUNK)],
                idx_v.at[b], sem_i[b]).wait()

        def scale_idx(b):
            @plsc.parallel_loop(0, CHUNK // 16, unroll=4)
            def _(j):
                off = j << 4
                idx_v[b, pl.ds(off, 16)] = idx_v[b, pl.ds(off, 16)] << 3

        def fire_gather(ci, b):
            pltpu.async_copy(w_hbm.at[idx_v.at[b]], rows_v.at[b], sem_g[b])

        def drain_gather(ci, b):
            pltpu.make_async_copy(
                w_hbm.at[idx_v.at[b]], rows_v.at[b], sem_g[b]).wait()

        def _out_slices(ci, b):
            tau = (base >> 10) + ci
            ht = tau >> 7
            bt = tau & 127
            pairs = []
            for hi in range(gpc):
                for dt in range(2):
                    pairs.append((t_v.at[b, hi, dt],
                                  out_hbm.at[ht * 8 + hi, dt, bt]))
            return pairs

        def fire_out(ci, b):
            for src, dst in _out_slices(ci, b):
                pltpu.async_copy(src, dst, sem_o[b])

        def drain_out(ci, b):
            for src, dst in _out_slices(ci, b):
                pltpu.make_async_copy(src, dst, sem_o[b]).wait()

        col_vecs = [jnp.full((16,), d, jnp.int32) for d in range(D)]

        def transpose_chunk(b):
            g_ref = rows_v.at[b]

            @plsc.parallel_loop(0, CHUNK // 16, unroll=2)
            def _(j):
                rows = (j << 4) + lanes
                hi = j >> 3
                bi0 = (j & 7) << 4
                for d in range(D):
                    v = plsc.load_gather(g_ref, [rows, col_vecs[d]])
                    t_v[b, hi, d >> 3, d & 7, pl.ds(bi0, 16)] = v

        fire_idx(0, 0)
        fire_idx(1, 1)
        drain_idx(0, 0)
        scale_idx(0)
        fire_gather(0, 0)

        n_iters = n_chunks + 2
        assert n_iters % NBUF == 0

        def ring_body(c0):
            for u in range(NBUF):
                ci = c0 + u
                b0 = u
                b1 = (u + 1) % NBUF
                b2 = (u + 2) % NBUF

                @pl.when(ci + 2 < n_chunks)
                def _():
                    fire_idx(ci + 2, b2)

                @pl.when(ci + 1 < n_chunks)
                def _():
                    drain_idx(ci + 1, b1)
                    scale_idx(b1)
                    fire_gather(ci + 1, b1)

                @pl.when(ci >= 2)
                def _():
                    drain_out(ci - 2, b1)

                @pl.when(ci < n_chunks)
                def _():
                    drain_gather(ci, b0)
                    transpose_chunk(b0)
                    fire_out(ci, b0)

        pl.loop(0, n_iters, step=NBUF, unroll=False)(ring_body)

    return k(xf, Wc)


def kernel(x, W):
    b, h = x.shape
    n = b * h
    x4 = x.reshape(b // 128, 128, h // 8, 8)
    xf = jnp.transpose(x4, (2, 0, 3, 1)).reshape(n).astype(jnp.int32)
    wc = _tc_clamp_t(jnp.transpose(W)).reshape(TERMS * 8, D)
    p5 = _sc_gather_t(xf, wc, n)
    return jnp.transpose(p5, (2, 4, 0, 1, 3)).reshape(b, h, D)


# speedup vs baseline: 1.4678x; 1.0738x over previous
import functools

import jax
import jax.numpy as jnp
from jax import lax
from jax.experimental import pallas as pl
from jax.experimental.pallas import tpu as pltpu
from jax.experimental.pallas import tpu_sc as plsc

TERMS = 1000000
D = 16
NC = 2
NS = 16
NW = NC * NS
CHUNK = 1024
NBUF = 3
BLK = 16384


def _tc_clamp_t(Wt):
    grid = (TERMS + BLK - 1) // BLK

    def body(w_ref, o_ref):
        o_ref[:, :D] = jnp.clip(w_ref[...].T, 0.0, 1.0)

    return pl.pallas_call(
        body,
        grid=(grid,),
        in_specs=[pl.BlockSpec((D, BLK), lambda i: (0, i))],
        out_specs=pl.BlockSpec((BLK, 128), lambda i: (i, 0)),
        out_shape=jax.ShapeDtypeStruct((TERMS, 128), jnp.float32),
    )(Wt)


def _sc_gather_t(xf, Wc, n):
    rows_per_w = n // NW
    n_chunks = rows_per_w // CHUNK
    nh = n // 16384
    gpc = CHUNK // 128

    mesh = plsc.VectorSubcoreMesh(core_axis_name="c", subcore_axis_name="s")

    @functools.partial(
        pl.kernel,
        mesh=mesh,
        compiler_params=pltpu.CompilerParams(
            use_tc_tiling_on_sc=False, needs_layout_passes=False),
        out_type=jax.ShapeDtypeStruct((nh, 2, 128, 8, 128), jnp.float32),
        scratch_types=[
            pltpu.VMEM((NBUF, CHUNK), jnp.int32),
            pltpu.VMEM((NBUF, CHUNK, D), jnp.float32),
            pltpu.VMEM((NBUF, gpc, 2, 8, 128), jnp.float32),
        ]
        + [pltpu.SemaphoreType.DMA] * (3 * NBUF),
    )
    def k(x_hbm, w_hbm, out_hbm, idx_v, rows_v, t_v, *sems):
        sem_i = sems[0:NBUF]
        sem_g = sems[NBUF:2 * NBUF]
        sem_o = sems[2 * NBUF:3 * NBUF]
        wid = lax.axis_index("s") * NC + lax.axis_index("c")
        base = wid * rows_per_w
        lanes = lax.iota(jnp.int32, 16)

        def fire_idx(ci, b):
            pltpu.async_copy(
                x_hbm.at[pl.ds(base + ci * CHUNK, CHUNK)],
                idx_v.at[b], sem_i[b])

        def drain_idx(ci, b):
            pltpu.make_async_copy(
                x_hbm.at[pl.ds(base + ci * CHUNK, CHUNK)],
                idx_v.at[b], sem_i[b]).wait()

        def scale_idx(b):
            @plsc.parallel_loop(0, CHUNK // 16, unroll=4)
            def _(j):
                off = j << 4
                idx_v[b, pl.ds(off, 16)] = idx_v[b, pl.ds(off, 16)] << 3

        def fire_gather(ci, b):
            pltpu.async_copy(w_hbm.at[idx_v.at[b]], rows_v.at[b], sem_g[b])

        def drain_gather(ci, b):
            pltpu.make_async_copy(
                w_hbm.at[idx_v.at[b]], rows_v.at[b], sem_g[b]).wait()

        def _out_slices(ci, b):
            tau = (base >> 10) + ci
            ht = tau >> 7
            bt = tau & 127
            pairs = []
            for hi in range(gpc):
                for dt in range(2):
                    pairs.append((t_v.at[b, hi, dt],
                                  out_hbm.at[ht * 8 + hi, dt, bt]))
            return pairs

        def fire_out(ci, b):
            for src, dst in _out_slices(ci, b):
                pltpu.async_copy(src, dst, sem_o[b])

        def drain_out(ci, b):
            for src, dst in _out_slices(ci, b):
                pltpu.make_async_copy(src, dst, sem_o[b]).wait()

        col_vecs = [jnp.full((16,), d, jnp.int32) for d in range(D)]

        def transpose_chunk(b):
            g_ref = rows_v.at[b]

            @plsc.parallel_loop(0, CHUNK // 16, unroll=2)
            def _(j):
                rows = (j << 4) + lanes
                hi = j >> 3
                bi0 = (j & 7) << 4
                for d in range(D):
                    v = plsc.load_gather(g_ref, [rows, col_vecs[d]])
                    t_v[b, hi, d >> 3, d & 7, pl.ds(bi0, 16)] = v

        fire_idx(0, 0)
        fire_idx(1, 1)
        drain_idx(0, 0)
        scale_idx(0)
        fire_gather(0, 0)

        n_iters = n_chunks + 2
        assert n_iters % NBUF == 0

        def ring_body(c0):
            for u in range(NBUF):
                ci = c0 + u
                b0 = u
                b1 = (u + 1) % NBUF
                b2 = (u + 2) % NBUF

                @pl.when(ci + 2 < n_chunks)
                def _():
                    fire_idx(ci + 2, b2)

                @pl.when(ci + 1 < n_chunks)
                def _():
                    drain_idx(ci + 1, b1)
                    scale_idx(b1)
                    fire_gather(ci + 1, b1)

                @pl.when(ci >= 2)
                def _():
                    drain_out(ci - 2, b1)

                @pl.when(ci < n_chunks)
                def _():
                    drain_gather(ci, b0)
                    transpose_chunk(b0)
                    fire_out(ci, b0)

        pl.loop(0, n_iters, step=NBUF, unroll=False)(ring_body)

    return k(xf, Wc)


def kernel(x, W):
    b, h = x.shape
    n = b * h
    x4 = x.reshape(b // 128, 128, h // 8, 8)
    xf = jnp.transpose(x4, (2, 0, 3, 1)).reshape(n).astype(jnp.int32)
    wc = _tc_clamp_t(jnp.transpose(W)).reshape(TERMS * 8, D)
    p5 = _sc_gather_t(xf, wc, n)
    return jnp.transpose(p5, (2, 4, 0, 1, 3)).reshape(b, h, D)
